# baseline (speedup 1.0000x reference)
import functools

import jax
import jax.numpy as jnp
from jax.experimental import pallas as pl
from jax.experimental.pallas import tpu as pltpu

N = 10000
E = 2000
F_IN = 256
HID = 256
F_OUT = 256

TN = 400
TE = 400

_INV_SCALE = 1.0 / 16.0


def _hf_kernel(h_ref, x_ref, o_ref):
    i = pl.program_id(0)
    part = jax.lax.dot_general(
        h_ref[...], x_ref[...], (((0,), (0,)), ((), ())),
        preferred_element_type=jnp.float32)

    @pl.when(i == 0)
    def _():
        o_ref[...] = part

    @pl.when(i > 0)
    def _():
        o_ref[...] = o_ref[...] + part


def _edge_attn_kernel(hf_ref, wq_ref, bq_ref, wk_ref, bk_ref, wv_ref, bv_ref,
                      wnk_ref, bnk_ref, att_ref, kn_ref, k_s, v_s):
    i = pl.program_id(0)

    @pl.when(i == 0)
    def _():
        hf = hf_ref[...]
        k_s[...] = jnp.dot(hf, wk_ref[...],
                           preferred_element_type=jnp.float32) + bk_ref[...]
        v_s[...] = jnp.dot(hf, wv_ref[...],
                           preferred_element_type=jnp.float32) + bv_ref[...]

    q = jnp.dot(hf_ref[pl.ds(i * TE, TE), :], wq_ref[...],
                preferred_element_type=jnp.float32) + bq_ref[...]
    s = jax.lax.dot_general(
        q, k_s[...], (((1,), (1,)), ((), ())),
        preferred_element_type=jnp.float32) * _INV_SCALE
    m = jnp.max(s, axis=-1, keepdims=True)
    e = jnp.exp(s - m)
    w = e / jnp.sum(e, axis=-1, keepdims=True)
    att = jnp.dot(w, v_s[...], preferred_element_type=jnp.float32)
    att_ref[...] = att
    kn_ref[...] = jnp.dot(att, wnk_ref[...],
                          preferred_element_type=jnp.float32) + bnk_ref[...]


def _node_kernel(x_ref, h_ref, att_ref, kn_ref, wnq_ref, bnq_ref,
                 wt_ref, bt_ref, o_ref):
    qn = jnp.dot(x_ref[...], wnq_ref[...],
                 preferred_element_type=jnp.float32) + bnq_ref[...]
    s = jax.lax.dot_general(
        qn, kn_ref[...], (((1,), (1,)), ((), ())),
        preferred_element_type=jnp.float32) * _INV_SCALE
    m = jnp.max(s, axis=-1, keepdims=True)
    e = jnp.exp(s - m)
    w = e / jnp.sum(e, axis=-1, keepdims=True)
    h_att = h_ref[...] * w
    agg = jnp.dot(h_att, att_ref[...], preferred_element_type=jnp.float32)
    o_ref[...] = jnp.dot(agg, wt_ref[...],
                         preferred_element_type=jnp.float32) + bt_ref[...]


@jax.jit
def kernel(X, H_norm, Wq, bq, Wk, bk, Wv, bv, Wnq, bnq, Wnk, bnk, Wt, bt):
    f32 = jnp.float32
    bq2, bk2, bv2 = bq.reshape(1, HID), bk.reshape(1, HID), bv.reshape(1, HID)
    bnq2, bnk2, bt2 = bnq.reshape(1, HID), bnk.reshape(1, HID), bt.reshape(1, F_OUT)

    full = lambda shape: pl.BlockSpec(shape, lambda i: (0, 0))

    hf = pl.pallas_call(
        _hf_kernel,
        grid=(N // TN,),
        in_specs=[
            pl.BlockSpec((TN, E), lambda i: (i, 0)),
            pl.BlockSpec((TN, F_IN), lambda i: (i, 0)),
        ],
        out_specs=full((E, F_IN)),
        out_shape=jax.ShapeDtypeStruct((E, F_IN), f32),
        compiler_params=pltpu.CompilerParams(
            dimension_semantics=("arbitrary",)),
    )(H_norm, X)

    attended, kn = pl.pallas_call(
        _edge_attn_kernel,
        grid=(E // TE,),
        in_specs=[
            full((E, F_IN)),
            full((F_IN, HID)), full((1, HID)),
            full((F_IN, HID)), full((1, HID)),
            full((F_IN, HID)), full((1, HID)),
            full((HID, HID)), full((1, HID)),
        ],
        out_specs=[
            pl.BlockSpec((TE, HID), lambda i: (i, 0)),
            pl.BlockSpec((TE, HID), lambda i: (i, 0)),
        ],
        out_shape=[
            jax.ShapeDtypeStruct((E, HID), f32),
            jax.ShapeDtypeStruct((E, HID), f32),
        ],
        scratch_shapes=[
            pltpu.VMEM((E, HID), f32),
            pltpu.VMEM((E, HID), f32),
        ],
        compiler_params=pltpu.CompilerParams(
            dimension_semantics=("arbitrary",)),
    )(hf, Wq, bq2, Wk, bk2, Wv, bv2, Wnk, bnk2)

    out = pl.pallas_call(
        _node_kernel,
        grid=(N // TN,),
        in_specs=[
            pl.BlockSpec((TN, F_IN), lambda i: (i, 0)),
            pl.BlockSpec((TN, E), lambda i: (i, 0)),
            full((E, HID)),
            full((E, HID)),
            full((F_IN, HID)), full((1, HID)),
            full((HID, F_OUT)), full((1, F_OUT)),
        ],
        out_specs=pl.BlockSpec((TN, F_OUT), lambda i: (i, 0)),
        out_shape=jax.ShapeDtypeStruct((N, F_OUT), f32),
        compiler_params=pltpu.CompilerParams(
            dimension_semantics=("parallel",)),
    )(X, H_norm, attended, kn, Wnq, bnq2, Wt, bt2)

    return out
